# baseline (speedup 1.0000x reference)
import functools

import jax
import jax.numpy as jnp
from jax import lax
from jax.experimental import pallas as pl
from jax.experimental.pallas import tpu as pltpu
from jax.experimental.pallas import tpu_sc as plsc

NC = 2
NS = 16
NW = NC * NS


def _sc_mesh():
    return plsc.VectorSubcoreMesh(
        core_axis_name="c", subcore_axis_name="s", num_cores=NC, num_subcores=NS
    )


def _sc_edge_layer(T, ea, src, dst, wa, ba, zeros, chunk, H, with_count):
    E = src.shape[0]
    D = T.shape[1]
    Np = zeros.shape[0]
    per_w = E // NW
    n_full = per_w // chunk
    rem = per_w - n_full * chunk
    rows_per_tile = Np // NS
    nterm = H // 16
    assert chunk % 8 == 0 and per_w % 8 == 0 and rem % 8 == 0
    assert rows_per_tile % 8 == 0 and D == 11 * H

    @functools.partial(
        pl.kernel,
        out_type=jax.ShapeDtypeStruct((NC, Np, 32), jnp.float32),
        mesh=_sc_mesh(),
        scratch_types=[
            pltpu.VMEM((chunk,), jnp.int32),
            pltpu.VMEM((chunk,), jnp.int32),
            pltpu.VMEM((chunk + 16,), jnp.float32),
            pltpu.VMEM((chunk, D), jnp.float32),
            pltpu.VMEM((chunk, 32), jnp.float32),
            pltpu.VMEM((16,), jnp.float32),
            pltpu.VMEM((16,), jnp.float32),
            pltpu.VMEM_SHARED((Np, 32), jnp.float32),
            pltpu.SemaphoreType.DMA,
        ],
        compiler_params=pltpu.CompilerParams(use_tc_tiling_on_sc=False),
    )
    def k(T_hbm, ea_hbm, src_hbm, dst_hbm, wa_hbm, ba_hbm, zeros_hbm, out_hbm,
          sidx_v, didx_v, a_v, rows_v, m_v, wa_v, ba_v, acc_sh, sem):
        cid = lax.axis_index("c")
        sid = lax.axis_index("s")
        wid = sid * NC + cid
        base = wid * per_w
        r0 = sid * rows_per_tile
        pltpu.sync_copy(wa_hbm, wa_v)
        pltpu.sync_copy(ba_hbm, ba_v)
        pltpu.sync_copy(zeros_hbm.at[pl.ds(r0, rows_per_tile)],
                        acc_sh.at[pl.ds(r0, rows_per_tile)])
        plsc.subcore_barrier()

        cvec = jnp.where(lax.iota(jnp.int32, 16) == 0, 1.0, 0.0)
        zvec = jnp.zeros((16,), jnp.float32)
        wa_vec = wa_v[...]
        ba_vec = ba_v[...]

        def combine(e, _):
            a = a_v[pl.ds(e, 16)][0]
            accs = [rows_v[e, pl.ds(10 * H + t * 16, 16)] for t in range(nterm)]
            for kk in range(10):
                c = jnp.maximum(a * wa_vec[kk] + ba_vec[kk], 0.0)
                for t in range(nterm):
                    accs[t] = accs[t] + c * rows_v[e, pl.ds(kk * H + t * 16, 16)]
            for t in range(nterm):
                m_v[e, pl.ds(t * 16, 16)] = accs[t]
            if with_count:
                m_v[e, pl.ds(16, 16)] = cvec
            return 0

        def zero_m(e, _):
            m_v[e, pl.ds(0, 16)] = zvec
            m_v[e, pl.ds(16, 16)] = zvec
            return 0

        def do_chunk(off, size):
            pltpu.sync_copy(src_hbm.at[pl.ds(off, size)],
                            sidx_v if size == chunk else sidx_v.at[pl.ds(0, size)])
            pltpu.sync_copy(ea_hbm.at[pl.ds(off, size)], a_v.at[pl.ds(0, size)])
            pltpu.async_copy(T_hbm.at[sidx_v], rows_v, sem).wait()
            lax.fori_loop(0, size, combine, 0)
            if size < chunk:
                lax.fori_loop(size, chunk, zero_m, 0)
            pltpu.sync_copy(dst_hbm.at[pl.ds(off, size)],
                            didx_v if size == chunk else didx_v.at[pl.ds(0, size)])
            pltpu.sync_copy(m_v, acc_sh.at[didx_v], add=True)

        for j in range(n_full):
            do_chunk(base + j * chunk, chunk)
        if rem:
            do_chunk(base + n_full * chunk, rem)

        plsc.subcore_barrier()
        pltpu.sync_copy(acc_sh.at[pl.ds(r0, rows_per_tile)],
                        out_hbm.at[cid, pl.ds(r0, rows_per_tile)])

    return k(T, ea, src, dst, wa, ba, zeros)


def _elu(v):
    return jnp.where(v > 0, v, jnp.exp(jnp.minimum(v, 0.0)) - 1.0)


def _tc_pre(x, Wfull, DT):
    N = x.shape[0]
    DR = Wfull.shape[1] - DT

    def body(x_ref, w_ref, t_ref, xr_ref):
        P = jnp.dot(x_ref[...], w_ref[...], preferred_element_type=jnp.float32)
        t_ref[...] = P[:, 0:DT]
        xr_ref[...] = P[:, DT:]

    return pl.pallas_call(
        body,
        out_shape=(jax.ShapeDtypeStruct((N, DT), jnp.float32),
                   jax.ShapeDtypeStruct((N, DR), jnp.float32)),
    )(x, Wfull)


def _tc_mid(xr, aggA, aggB, bias1, Wcat2, root2, H):
    N = xr.shape[0]

    def body(xr_ref, a_ref, b_ref, b1_ref, w2_ref, r2_ref,
             t2_ref, h1r_ref, cnt_ref):
        s = a_ref[:, 0:H] + b_ref[:, 0:H]
        cnt = a_ref[:, H:H + 1] + b_ref[:, H:H + 1]
        h1 = _elu(xr_ref[...] + s / jnp.maximum(cnt, 1.0) + b1_ref[...])
        t2_ref[...] = jnp.dot(h1, w2_ref[...], preferred_element_type=jnp.float32)
        h1r_ref[...] = jnp.dot(h1, r2_ref[...], preferred_element_type=jnp.float32)
        cnt_ref[...] = cnt

    return pl.pallas_call(
        body,
        out_shape=(jax.ShapeDtypeStruct((N, Wcat2.shape[1]), jnp.float32),
                   jax.ShapeDtypeStruct((N, root2.shape[1]), jnp.float32),
                   jax.ShapeDtypeStruct((N, 1), jnp.float32)),
    )(xr, aggA, aggB, bias1, Wcat2, root2)


def _tc_head(h1r, cnt, aggA, aggB, bias2, batch2d, Wfc1, bfc1, Wfc2, bfc2,
             n_graphs, H):
    N = h1r.shape[0]
    n_cls = Wfc2.shape[1]

    def body(h1r_ref, cnt_ref, a_ref, b_ref, bias2_ref, batch_ref,
             w1_ref, c1_ref, w2_ref, c2_ref, out_ref):
        s = a_ref[:, 0:H] + b_ref[:, 0:H]
        agg = s / jnp.maximum(cnt_ref[...], 1.0)
        h2 = _elu(h1r_ref[...] + agg + bias2_ref[...])
        gids = lax.broadcasted_iota(jnp.int32, (n_graphs, N), 0)
        mask = (batch_ref[...] == gids).astype(jnp.float32)
        pooled = jnp.dot(mask, h2, preferred_element_type=jnp.float32)
        cg = jnp.maximum(jnp.sum(mask, axis=1, keepdims=True), 1.0)
        pooled = pooled / cg
        z = _elu(jnp.dot(pooled, w1_ref[...], preferred_element_type=jnp.float32)
                 + c1_ref[...])
        logits = (jnp.dot(z, w2_ref[...], preferred_element_type=jnp.float32)
                  + c2_ref[...])
        mx = jnp.max(logits, axis=1, keepdims=True)
        lse = jnp.log(jnp.sum(jnp.exp(logits - mx), axis=1, keepdims=True)) + mx
        out_ref[...] = logits - lse

    return pl.pallas_call(
        body,
        out_shape=jax.ShapeDtypeStruct((n_graphs, n_cls), jnp.float32),
    )(h1r, cnt, aggA, aggB, bias2, batch2d, Wfc1, bfc1, Wfc2, bfc2)


def kernel(x, edge_index, edge_attr, batch, W1a, b1a, W1b, b1b, root1, bias1,
           W2a, b2a, W2b, b2b, root2, bias2, Wfc1, bfc1, Wfc2, bfc2):
    N, F = x.shape
    E = edge_attr.shape[0]
    H1 = root1.shape[1]
    H2 = root2.shape[1]
    n_graphs = 64

    src = edge_index[0]
    dst = edge_index[1]
    ea = edge_attr.reshape(E)

    def cat_table(Wb, bb, fin, h):
        B = jnp.concatenate([Wb.reshape(10, fin, h),
                             bb.reshape(1, fin, h)], axis=0)
        return B.transpose(1, 0, 2).reshape(fin, 11 * h)

    Wfull1 = jnp.concatenate([cat_table(W1b, b1b, F, H1), root1], axis=1)
    Wcat2 = cat_table(W2b, b2b, H1, H2)
    wa1 = jnp.pad(W1a[0], (0, 6))
    ba1 = jnp.pad(b1a, (0, 6))
    wa2 = jnp.pad(W2a[0], (0, 6))
    ba2 = jnp.pad(b2a, (0, 6))

    n_pad = ((N + NS * 8 - 1) // (NS * 8)) * (NS * 8)
    zeros = jnp.zeros((n_pad, 32), jnp.float32)

    T1, xr = _tc_pre(x, Wfull1, 11 * H1)
    agg1 = _sc_edge_layer(T1, ea, src, dst, wa1, ba1, zeros,
                          chunk=200, H=H1, with_count=True)
    T2, h1r, cnt = _tc_mid(xr, agg1[0, :N], agg1[1, :N],
                           bias1.reshape(1, H1), Wcat2, root2, H1)
    agg2 = _sc_edge_layer(T2, ea, src, dst, wa2, ba2, zeros,
                          chunk=104, H=H2, with_count=False)
    return _tc_head(h1r, cnt, agg2[0, :N], agg2[1, :N], bias2.reshape(1, H2),
                    batch.reshape(1, N).astype(jnp.int32),
                    Wfc1, bfc1.reshape(1, -1), Wfc2, bfc2.reshape(1, -1),
                    n_graphs, H2)
